# Initial kernel scaffold; baseline (speedup 1.0000x reference)
import jax
import jax.numpy as jnp
from jax.experimental import pallas as pl


def kernel(z, embedding):
    raise NotImplementedError("write your pallas kernel here")


# speedup vs baseline: 3.3265x; 3.3265x over previous
import functools

import jax
import jax.numpy as jnp
from jax.experimental import pallas as pl
from jax.experimental.pallas import tpu as pltpu


def _vq_block_kernel(z_ref, emb_ref, embt_ref, zq_ref, idx_ref, loss_ref):
    zb = z_ref[0]
    emb = emb_ref[...]
    k_dim = emb.shape[0]
    t_blk = zb.shape[1]

    scores = jnp.dot(emb, zb, preferred_element_type=jnp.float32)
    e2 = jnp.sum(emb * emb, axis=1)
    dist = e2[:, None] - 2.0 * scores

    idx = jnp.argmin(dist, axis=0)
    vals = jnp.min(dist, axis=0)
    z2 = jnp.sum(zb * zb, axis=0)

    onehot = (jax.lax.broadcasted_iota(jnp.int32, (k_dim, t_blk), 0)
              == idx[None, :]).astype(jnp.float32)
    zq_ref[0] = jnp.dot(embt_ref[...], onehot,
                        preferred_element_type=jnp.float32)
    idx_ref[0, 0] = idx

    @pl.when(jnp.logical_and(pl.program_id(0) == 0, pl.program_id(1) == 0))
    def _init():
        loss_ref[...] = jnp.zeros((1, 1), jnp.float32)

    part = jnp.sum((vals + z2).reshape(1, t_blk), axis=1, keepdims=True)
    loss_ref[...] += part


@jax.jit
def kernel(z, embedding):
    B, D, T = z.shape
    K = embedding.shape[0]
    t_blk = 512
    nt = T // t_blk

    grid = (B, nt)
    zq, idx3, loss_raw = pl.pallas_call(
        _vq_block_kernel,
        grid=grid,
        in_specs=[
            pl.BlockSpec((1, D, t_blk), lambda b, t: (b, 0, t)),
            pl.BlockSpec((K, D), lambda b, t: (0, 0)),
            pl.BlockSpec((D, K), lambda b, t: (0, 0)),
        ],
        out_specs=[
            pl.BlockSpec((1, D, t_blk), lambda b, t: (b, 0, t)),
            pl.BlockSpec((1, 1, t_blk), lambda b, t: (b * nt + t, 0, 0)),
            pl.BlockSpec((1, 1), lambda b, t: (0, 0)),
        ],
        out_shape=[
            jax.ShapeDtypeStruct((B, D, T), jnp.float32),
            jax.ShapeDtypeStruct((B * nt, 1, t_blk), jnp.int32),
            jax.ShapeDtypeStruct((1, 1), jnp.float32),
        ],
        compiler_params=pltpu.CompilerParams(
            dimension_semantics=("arbitrary", "arbitrary"),
        ),
    )(z, embedding, embedding.T)

    loss = loss_raw[0, 0] * (0.25 / (B * T * D))
    indices = idx3.reshape(B, T)
    return zq, loss, indices
